# baseline (speedup 1.0000x reference)
import functools

import jax
import jax.numpy as jnp
from jax import lax
from jax.experimental import pallas as pl
from jax.experimental.pallas import tpu as pltpu
from jax.experimental.pallas import tpu_sc as plsc

H, W, FD = 1024, 1024, 2
W2 = W + 1
NC, NS, L = 2, 16, 16
NW = NC * NS

N = 16384 * 200
PER_W = N // NW
C = 2048
CHUNKS = PER_W // C

PV = 16 * 66560
STAGE = PV // NS


def _sc_body(coords_hbm, table_hbm, out_hbm,
             shared, coords_v, i00_v, i01_v, i10_v, i11_v, wx_v, wy_v,
             r00_v, r01_v, r10_v, r11_v, out_v, sem):
    cid = lax.axis_index("c")
    sid = lax.axis_index("s")
    wid = sid * NC + cid
    base_coord = wid * PER_W

    pltpu.sync_copy(table_hbm.at[pl.ds(sid * STAGE, STAGE)],
                    shared.at[pl.ds(sid * STAGE, STAGE)])
    plsc.subcore_barrier()

    def chunk_body(g, carry):
        cbase = base_coord + g * C
        fbase = cbase * 2
        pltpu.sync_copy(coords_hbm.at[pl.ds(fbase, 2 * C)], coords_v)

        def idx_body(i, carry2):
            iota = lax.iota(jnp.int32, L)
            xi = iota * 2 + i * (2 * L)
            x = plsc.load_gather(coords_v, [xi]) * float(W - 1)
            y = plsc.load_gather(coords_v, [xi + 1]) * float(H - 1)
            x0 = x.astype(jnp.int32)
            y0 = y.astype(jnp.int32)
            wx = x - x0.astype(jnp.float32)
            wy = y - y0.astype(jnp.float32)
            idx = y0 * W2 + x0
            b = i * L
            i00_v[pl.ds(b, L)] = idx
            i01_v[pl.ds(b, L)] = idx + 1
            i10_v[pl.ds(b, L)] = idx + W2
            i11_v[pl.ds(b, L)] = idx + (W2 + 1)
            wx_v[pl.ds(b, L)] = wx
            wy_v[pl.ds(b, L)] = wy
            return carry2

        lax.fori_loop(0, C // L, idx_body, 0)

        cps = [
            pltpu.async_copy(shared.at[i00_v], r00_v, sem),
            pltpu.async_copy(shared.at[i01_v], r01_v, sem),
            pltpu.async_copy(shared.at[i10_v], r10_v, sem),
            pltpu.async_copy(shared.at[i11_v], r11_v, sem),
        ]
        for cp in cps:
            cp.wait()

        def mix_body(i, carry2):
            b = i * L
            iota = lax.iota(jnp.int32, L)
            wx = wx_v[pl.ds(b, L)]
            wy = wy_v[pl.ds(b, L)]
            u00 = plsc.bitcast(r00_v[pl.ds(b, L)], jnp.int32)
            u01 = plsc.bitcast(r01_v[pl.ds(b, L)], jnp.int32)
            u10 = plsc.bitcast(r10_v[pl.ds(b, L)], jnp.int32)
            u11 = plsc.bitcast(r11_v[pl.ds(b, L)], jnp.int32)
            hm = jnp.int32(-65536)
            a00 = plsc.bitcast(lax.shift_left(u00, 16), jnp.float32)
            a01 = plsc.bitcast(lax.shift_left(u01, 16), jnp.float32)
            a10 = plsc.bitcast(lax.shift_left(u10, 16), jnp.float32)
            a11 = plsc.bitcast(lax.shift_left(u11, 16), jnp.float32)
            b00 = plsc.bitcast(u00 & hm, jnp.float32)
            b01 = plsc.bitcast(u01 & hm, jnp.float32)
            b10 = plsc.bitcast(u10 & hm, jnp.float32)
            b11 = plsc.bitcast(u11 & hm, jnp.float32)
            t0 = a00 + wx * (a01 - a00)
            u0 = a10 + wx * (a11 - a10)
            o0 = t0 + wy * (u0 - t0)
            t1 = b00 + wx * (b01 - b00)
            u1 = b10 + wx * (b11 - b10)
            o1 = t1 + wy * (u1 - t1)
            pos = iota * 2 + (2 * b)
            plsc.store_scatter(out_v, [pos], o0)
            plsc.store_scatter(out_v, [pos + 1], o1)
            return carry2

        lax.fori_loop(0, C // L, mix_body, 0)

        pltpu.sync_copy(out_v, out_hbm.at[pl.ds(fbase, 2 * C)])
        return carry

    lax.fori_loop(0, CHUNKS, chunk_body, 0)


_sc_sample = functools.partial(
    pl.kernel,
    out_type=jax.ShapeDtypeStruct((N * FD,), jnp.float32),
    mesh=plsc.VectorSubcoreMesh(
        core_axis_name="c", subcore_axis_name="s", num_cores=NC, num_subcores=NS
    ),
    compiler_params=pltpu.CompilerParams(needs_layout_passes=False),
    scratch_types=[
        pltpu.VMEM_SHARED((PV,), jnp.float32),
        pltpu.VMEM((2 * C,), jnp.float32),
        pltpu.VMEM((C,), jnp.int32),
        pltpu.VMEM((C,), jnp.int32),
        pltpu.VMEM((C,), jnp.int32),
        pltpu.VMEM((C,), jnp.int32),
        pltpu.VMEM((C,), jnp.float32),
        pltpu.VMEM((C,), jnp.float32),
        pltpu.VMEM((C,), jnp.float32),
        pltpu.VMEM((C,), jnp.float32),
        pltpu.VMEM((C,), jnp.float32),
        pltpu.VMEM((C,), jnp.float32),
        pltpu.VMEM((2 * C,), jnp.float32),
        pltpu.SemaphoreType.DMA,
    ],
)(_sc_body)


def kernel(coords, vector_field):
    g16 = lax.bitcast_convert_type(
        vector_field.astype(jnp.bfloat16), jnp.uint16
    ).astype(jnp.uint32)
    packed = g16[..., 0] | (g16[..., 1] << 16)
    packed = jnp.pad(packed, ((0, 1), (0, 1)), mode="edge").reshape(-1)
    packed = jnp.pad(packed, (0, PV - W2 * (H + 1)))
    table = lax.bitcast_convert_type(packed, jnp.float32)
    out = _sc_sample(coords.reshape(-1), table)
    return out.reshape(*coords.shape[:-1], FD)


# speedup vs baseline: 15.8584x; 15.8584x over previous
import functools

import jax
import jax.numpy as jnp
from jax import lax
from jax.experimental import pallas as pl
from jax.experimental.pallas import tpu as pltpu
from jax.experimental.pallas import tpu_sc as plsc

H, W, FD = 1024, 1024, 2
W2 = W + 1
NC, NS, L = 2, 16, 16
NW = NC * NS

N = 16384 * 200
NP = 200
NX = 16384
SEG = NX // NW

PV = 16 * 66560
STAGE = PV // NS


def _sc_body(coords_hbm, table_hbm, out_hbm,
             shared, x_v, y_v, i00_v, i01_v, i10_v, i11_v, wx_v, wy_v,
             r00_v, r01_v, r10_v, r11_v, o0_v, o1_v, sem):
    cid = lax.axis_index("c")
    sid = lax.axis_index("s")
    wid = sid * NC + cid
    nbase = wid * SEG

    pltpu.sync_copy(table_hbm.at[pl.ds(sid * STAGE, STAGE)],
                    shared.at[pl.ds(sid * STAGE, STAGE)])
    plsc.subcore_barrier()

    def plane_body(k, carry):
        base = k * (2 * NX) + nbase
        cpx = pltpu.async_copy(coords_hbm.at[pl.ds(base, SEG)], x_v, sem)
        cpy = pltpu.async_copy(coords_hbm.at[pl.ds(base + NX, SEG)], y_v, sem)
        cpx.wait()
        cpy.wait()

        def idx_body(i, carry2):
            b = i * L
            x = x_v[pl.ds(b, L)] * float(W - 1)
            y = y_v[pl.ds(b, L)] * float(H - 1)
            x0 = x.astype(jnp.int32)
            y0 = y.astype(jnp.int32)
            wx = x - x0.astype(jnp.float32)
            wy = y - y0.astype(jnp.float32)
            idx = y0 * W2 + x0
            i00_v[pl.ds(b, L)] = idx
            i01_v[pl.ds(b, L)] = idx + 1
            i10_v[pl.ds(b, L)] = idx + W2
            i11_v[pl.ds(b, L)] = idx + (W2 + 1)
            wx_v[pl.ds(b, L)] = wx
            wy_v[pl.ds(b, L)] = wy
            return carry2

        lax.fori_loop(0, SEG // L, idx_body, 0)

        cps = [
            pltpu.async_copy(shared.at[i00_v], r00_v, sem),
            pltpu.async_copy(shared.at[i01_v], r01_v, sem),
            pltpu.async_copy(shared.at[i10_v], r10_v, sem),
            pltpu.async_copy(shared.at[i11_v], r11_v, sem),
        ]
        for cp in cps:
            cp.wait()

        def mix_body(i, carry2):
            b = i * L
            wx = wx_v[pl.ds(b, L)]
            wy = wy_v[pl.ds(b, L)]
            u00 = plsc.bitcast(r00_v[pl.ds(b, L)], jnp.int32)
            u01 = plsc.bitcast(r01_v[pl.ds(b, L)], jnp.int32)
            u10 = plsc.bitcast(r10_v[pl.ds(b, L)], jnp.int32)
            u11 = plsc.bitcast(r11_v[pl.ds(b, L)], jnp.int32)
            hm = jnp.int32(-65536)
            a00 = plsc.bitcast(lax.shift_left(u00, 16), jnp.float32)
            a01 = plsc.bitcast(lax.shift_left(u01, 16), jnp.float32)
            a10 = plsc.bitcast(lax.shift_left(u10, 16), jnp.float32)
            a11 = plsc.bitcast(lax.shift_left(u11, 16), jnp.float32)
            b00 = plsc.bitcast(u00 & hm, jnp.float32)
            b01 = plsc.bitcast(u01 & hm, jnp.float32)
            b10 = plsc.bitcast(u10 & hm, jnp.float32)
            b11 = plsc.bitcast(u11 & hm, jnp.float32)
            t0 = a00 + wx * (a01 - a00)
            u0 = a10 + wx * (a11 - a10)
            o0 = t0 + wy * (u0 - t0)
            t1 = b00 + wx * (b01 - b00)
            u1 = b10 + wx * (b11 - b10)
            o1 = t1 + wy * (u1 - t1)
            o0_v[pl.ds(b, L)] = o0
            o1_v[pl.ds(b, L)] = o1
            return carry2

        lax.fori_loop(0, SEG // L, mix_body, 0)

        cpo0 = pltpu.async_copy(o0_v, out_hbm.at[pl.ds(base, SEG)], sem)
        cpo1 = pltpu.async_copy(o1_v, out_hbm.at[pl.ds(base + NX, SEG)], sem)
        cpo0.wait()
        cpo1.wait()
        return carry

    lax.fori_loop(0, NP, plane_body, 0)


_sc_sample = functools.partial(
    pl.kernel,
    out_type=jax.ShapeDtypeStruct((N * FD,), jnp.float32),
    mesh=plsc.VectorSubcoreMesh(
        core_axis_name="c", subcore_axis_name="s", num_cores=NC, num_subcores=NS
    ),
    compiler_params=pltpu.CompilerParams(needs_layout_passes=False),
    scratch_types=[
        pltpu.VMEM_SHARED((PV,), jnp.float32),
        pltpu.VMEM((SEG,), jnp.float32),
        pltpu.VMEM((SEG,), jnp.float32),
        pltpu.VMEM((SEG,), jnp.int32),
        pltpu.VMEM((SEG,), jnp.int32),
        pltpu.VMEM((SEG,), jnp.int32),
        pltpu.VMEM((SEG,), jnp.int32),
        pltpu.VMEM((SEG,), jnp.float32),
        pltpu.VMEM((SEG,), jnp.float32),
        pltpu.VMEM((SEG,), jnp.float32),
        pltpu.VMEM((SEG,), jnp.float32),
        pltpu.VMEM((SEG,), jnp.float32),
        pltpu.VMEM((SEG,), jnp.float32),
        pltpu.VMEM((SEG,), jnp.float32),
        pltpu.VMEM((SEG,), jnp.float32),
        pltpu.SemaphoreType.DMA,
    ],
)(_sc_body)


def kernel(coords, vector_field):
    vt = vector_field.transpose(0, 2, 1)
    g16 = lax.bitcast_convert_type(
        vt.astype(jnp.bfloat16), jnp.uint16
    ).astype(jnp.uint32)
    packed = g16[:, 0, :] | (g16[:, 1, :] << 16)
    packed = jnp.pad(packed, ((0, 1), (0, 1)), mode="edge").reshape(-1)
    packed = jnp.pad(packed, (0, PV - W2 * (H + 1)))
    table = lax.bitcast_convert_type(packed, jnp.float32)
    ct = coords.transpose(1, 2, 0).reshape(-1)
    out = _sc_sample(ct, table)
    return out.reshape(NP, FD, NX).transpose(2, 0, 1)


# speedup vs baseline: 31.6808x; 1.9977x over previous
import functools

import jax
import jax.numpy as jnp
from jax import lax
from jax.experimental import pallas as pl
from jax.experimental.pallas import tpu as pltpu
from jax.experimental.pallas import tpu_sc as plsc

H, W, FD = 1024, 1024, 2
W2 = W + 1
NC, NS, L = 2, 16, 16
NW = NC * NS

N = 16384 * 200
NP = 200
NX = 16384
SEG = NX // NW

PV = 16 * 66560
STAGE = PV // NS


def _idx_loop(x_v, y_v, i00_v, i01_v, i10_v, i11_v, wx_v, wy_v):
    def idx_body(i, carry):
        b = i * L
        x = x_v[pl.ds(b, L)] * float(W - 1)
        y = y_v[pl.ds(b, L)] * float(H - 1)
        x0 = x.astype(jnp.int32)
        y0 = y.astype(jnp.int32)
        wx = x - x0.astype(jnp.float32)
        wy = y - y0.astype(jnp.float32)
        idx = y0 * W2 + x0
        i00_v[pl.ds(b, L)] = idx
        i01_v[pl.ds(b, L)] = idx + 1
        i10_v[pl.ds(b, L)] = idx + W2
        i11_v[pl.ds(b, L)] = idx + (W2 + 1)
        wx_v[pl.ds(b, L)] = wx
        wy_v[pl.ds(b, L)] = wy
        return carry

    lax.fori_loop(0, SEG // L, idx_body, 0)


def _mix_loop(r00_v, r01_v, r10_v, r11_v, wx_v, wy_v, o0_v, o1_v):
    def mix_body(i, carry):
        b = i * L
        wx = wx_v[pl.ds(b, L)]
        wy = wy_v[pl.ds(b, L)]
        u00 = plsc.bitcast(r00_v[pl.ds(b, L)], jnp.int32)
        u01 = plsc.bitcast(r01_v[pl.ds(b, L)], jnp.int32)
        u10 = plsc.bitcast(r10_v[pl.ds(b, L)], jnp.int32)
        u11 = plsc.bitcast(r11_v[pl.ds(b, L)], jnp.int32)
        hm = jnp.int32(-65536)
        a00 = plsc.bitcast(lax.shift_left(u00, 16), jnp.float32)
        a01 = plsc.bitcast(lax.shift_left(u01, 16), jnp.float32)
        a10 = plsc.bitcast(lax.shift_left(u10, 16), jnp.float32)
        a11 = plsc.bitcast(lax.shift_left(u11, 16), jnp.float32)
        b00 = plsc.bitcast(u00 & hm, jnp.float32)
        b01 = plsc.bitcast(u01 & hm, jnp.float32)
        b10 = plsc.bitcast(u10 & hm, jnp.float32)
        b11 = plsc.bitcast(u11 & hm, jnp.float32)
        t0 = a00 + wx * (a01 - a00)
        u0 = a10 + wx * (a11 - a10)
        t1 = b00 + wx * (b01 - b00)
        u1 = b10 + wx * (b11 - b10)
        o0_v[pl.ds(b, L)] = t0 + wy * (u0 - t0)
        o1_v[pl.ds(b, L)] = t1 + wy * (u1 - t1)
        return carry

    lax.fori_loop(0, SEG // L, mix_body, 0)


def _sc_body(coords_hbm, table_hbm, out_hbm, shared,
             x0_v, y0_v, x1_v, y1_v,
             a00_v, a01_v, a10_v, a11_v, b00_v, b01_v, b10_v, b11_v,
             wxa_v, wya_v, wxb_v, wyb_v,
             p00_v, p01_v, p10_v, p11_v, q00_v, q01_v, q10_v, q11_v,
             oa0_v, oa1_v, ob0_v, ob1_v,
             si0, si1, sg0, sg1, so0, so1):
    cid = lax.axis_index("c")
    sid = lax.axis_index("s")
    wid = sid * NC + cid
    nbase = wid * SEG

    xs = [x0_v, x1_v]
    ys = [y0_v, y1_v]
    idxs = [[a00_v, a01_v, a10_v, a11_v], [b00_v, b01_v, b10_v, b11_v]]
    wxs = [wxa_v, wxb_v]
    wys = [wya_v, wyb_v]
    rs = [[p00_v, p01_v, p10_v, p11_v], [q00_v, q01_v, q10_v, q11_v]]
    o0s = [oa0_v, ob0_v]
    o1s = [oa1_v, ob1_v]
    sin = [si0, si1]
    sgat = [sg0, sg1]
    sout = [so0, so1]

    pltpu.sync_copy(table_hbm.at[pl.ds(sid * STAGE, STAGE)],
                    shared.at[pl.ds(sid * STAGE, STAGE)])
    plsc.subcore_barrier()

    def in_start(k, b):
        base = k * (2 * NX) + nbase
        pltpu.async_copy(coords_hbm.at[pl.ds(base, SEG)], xs[b], sin[b])
        pltpu.async_copy(coords_hbm.at[pl.ds(base + NX, SEG)], ys[b], sin[b])

    def in_wait(k, b):
        base = k * (2 * NX) + nbase
        pltpu.make_async_copy(coords_hbm.at[pl.ds(base, SEG)], xs[b], sin[b]).wait()
        pltpu.make_async_copy(coords_hbm.at[pl.ds(base + NX, SEG)], ys[b], sin[b]).wait()

    def gat_start(b):
        for iv, rv in zip(idxs[b], rs[b]):
            pltpu.async_copy(shared.at[iv], rv, sgat[b])

    def gat_wait(b):
        for iv, rv in zip(idxs[b], rs[b]):
            pltpu.make_async_copy(shared.at[iv], rv, sgat[b]).wait()

    def out_start(k, b):
        base = k * (2 * NX) + nbase
        pltpu.async_copy(o0s[b], out_hbm.at[pl.ds(base, SEG)], sout[b])
        pltpu.async_copy(o1s[b], out_hbm.at[pl.ds(base + NX, SEG)], sout[b])

    def out_wait(k, b):
        base = k * (2 * NX) + nbase
        pltpu.make_async_copy(o0s[b], out_hbm.at[pl.ds(base, SEG)], sout[b]).wait()
        pltpu.make_async_copy(o1s[b], out_hbm.at[pl.ds(base + NX, SEG)], sout[b]).wait()

    in_start(0, 0)

    def pipe_body(t, carry):
        for buf in (0, 1):
            k = t * 2 + buf
            nbuf = 1 - buf
            if buf == 0:
                in_start(k + 1, nbuf)
            else:
                @pl.when(k + 1 < NP)
                def _():
                    in_start(k + 1, nbuf)
            in_wait(k, buf)
            _idx_loop(xs[buf], ys[buf], *idxs[buf], wxs[buf], wys[buf])
            gat_start(buf)

            def tail():
                gat_wait(nbuf)

                @pl.when(k >= 3)
                def _():
                    out_wait(k - 3, nbuf)

                _mix_loop(*rs[nbuf], wxs[nbuf], wys[nbuf], o0s[nbuf], o1s[nbuf])
                out_start(k - 1, nbuf)

            if buf == 1:
                tail()
            else:
                @pl.when(k >= 1)
                def _():
                    tail()
        return carry

    lax.fori_loop(0, NP // 2, pipe_body, 0)

    gat_wait(1)
    out_wait(NP - 3, 1)
    _mix_loop(*rs[1], wxs[1], wys[1], o0s[1], o1s[1])
    out_start(NP - 1, 1)
    out_wait(NP - 2, 0)
    out_wait(NP - 1, 1)


_sc_sample = functools.partial(
    pl.kernel,
    out_type=jax.ShapeDtypeStruct((N * FD,), jnp.float32),
    mesh=plsc.VectorSubcoreMesh(
        core_axis_name="c", subcore_axis_name="s", num_cores=NC, num_subcores=NS
    ),
    compiler_params=pltpu.CompilerParams(needs_layout_passes=False),
    scratch_types=[
        pltpu.VMEM_SHARED((PV,), jnp.float32),
    ] + [pltpu.VMEM((SEG,), jnp.float32) for _ in range(4)]
    + [pltpu.VMEM((SEG,), jnp.int32) for _ in range(8)]
    + [pltpu.VMEM((SEG,), jnp.float32) for _ in range(4)]
    + [pltpu.VMEM((SEG,), jnp.float32) for _ in range(8)]
    + [pltpu.VMEM((SEG,), jnp.float32) for _ in range(4)]
    + [pltpu.SemaphoreType.DMA for _ in range(6)],
)(_sc_body)


def kernel(coords, vector_field):
    vt = vector_field.transpose(0, 2, 1)
    g16 = lax.bitcast_convert_type(
        vt.astype(jnp.bfloat16), jnp.uint16
    ).astype(jnp.uint32)
    packed = g16[:, 0, :] | (g16[:, 1, :] << 16)
    packed = jnp.pad(packed, ((0, 1), (0, 1)), mode="edge").reshape(-1)
    packed = jnp.pad(packed, (0, PV - W2 * (H + 1)))
    table = lax.bitcast_convert_type(packed, jnp.float32)
    ct = coords.transpose(1, 2, 0).reshape(-1)
    out = _sc_sample(ct, table)
    return out.reshape(NP, FD, NX).transpose(2, 0, 1)
